# baseline (speedup 1.0000x reference)
import functools

import jax
import jax.numpy as jnp
from jax import lax
from jax.experimental import pallas as pl
from jax.experimental.pallas import tpu as pltpu
from jax.experimental.pallas import tpu_sc as plsc

_EMBED = 64
_BATCH = 16384
_NC, _NS = 2, 16
_NW = _NC * _NS
_BPW = _BATCH // _NW

_mesh = plsc.VectorSubcoreMesh(core_axis_name="c", subcore_axis_name="s")


@functools.partial(
    pl.kernel,
    out_type=jax.ShapeDtypeStruct((_NW, _BPW, _EMBED), jnp.float32),
    mesh=_mesh,
    scratch_types=[
        pltpu.VMEM((_BPW + 16,), jnp.int32),
        pltpu.VMEM((_BPW, _EMBED), jnp.float32),
        pltpu.SemaphoreType.DMA,
        pltpu.SemaphoreType.DMA,
        pltpu.SemaphoreType.DMA,
        pltpu.SemaphoreType.DMA,
    ],
)
def _sc_gather(idx_hbm, table_hbm, out_hbm, idx_v, buf_v, s0, s1, s2, s3):
    wid = lax.axis_index("s") * _NC + lax.axis_index("c")
    sems = (s0, s1, s2, s3)
    nsem = len(sems)
    pltpu.sync_copy(idx_hbm.at[wid], idx_v.at[pl.ds(0, _BPW)])

    def fire(g, carry):
        for k in range(nsem):
            b = g * nsem + k
            i = idx_v[pl.ds(b, 16)][0]
            pltpu.async_copy(table_hbm.at[i], buf_v.at[b], sems[k])
        return carry

    lax.fori_loop(0, _BPW // nsem, fire, 0, unroll=False)
    share = _BPW // nsem
    for k in range(nsem):
        pltpu.make_async_copy(
            table_hbm.at[pl.ds(0, share)],
            buf_v.at[pl.ds(k * share, share)],
            sems[k],
        ).wait()
    pltpu.sync_copy(buf_v, out_hbm.at[wid])


def kernel(inputs, table):
    idx = jnp.reshape(inputs.astype(jnp.int32), (_NW, _BPW))
    out = _sc_gather(idx, table)
    return jnp.reshape(out, (_BATCH, _EMBED))
